# baseline (speedup 1.0000x reference)
import jax
import jax.numpy as jnp
from jax import lax
from jax.experimental import pallas as pl
from jax.experimental.pallas import tpu as pltpu
from jax.experimental.pallas import tpu_sc as plsc

B, D, E = 1024, 2048, 64
NC, NS, L = 2, 16, 16
NW = NC * NS
BPW = B // NW
EC = 8
NEC = E // EC
SUB = 128
KPS = SUB // L


def _sc_body(fp_hbm, baset_hbm, deltat_hbm, out_hbm,
             base_v, out_v0, out_v1, fp_v, deltat_v):
    wid = lax.axis_index("s") * NC + lax.axis_index("c")
    b0 = wid * BPW
    bufs = (out_v0, out_v1)

    pltpu.sync_copy(deltat_hbm, deltat_v)

    def run(sem0, sem1):
        dma_sems = (sem0, sem1)

        def ec_body(ec, _):
            e0 = pl.multiple_of(ec * EC, EC)
            pltpu.sync_copy(baset_hbm.at[pl.ds(e0, EC), :], base_v)
            dvs = [deltat_v[ec * EC + e] for e in range(EC)]

            def pair_body(bi2, _, dvs=dvs):
                bb = b0 + bi2 * 2

                for j in range(2):
                    @pl.when((ec > 0) | (bi2 > 0))
                    def _(j=j):
                        pltpu.make_async_copy(
                            bufs[j], out_hbm.at[bb + j, pl.ds(e0, EC), :],
                            dma_sems[j],
                        ).wait()

                pltpu.sync_copy(fp_hbm.at[pl.ds(bb, 2), :], fp_v)

                for sub in range(D // SUB):
                    fj = [[fp_v[j, pl.ds(sub * SUB + k * L, L)]
                           .astype(jnp.float32) for k in range(KPS)]
                          for j in range(2)]
                    for e in range(EC):
                        for k in range(KPS):
                            off = sub * SUB + k * L
                            bv = base_v[e, pl.ds(off, L)]
                            out_v0[e, pl.ds(off, L)] = bv + fj[0][k] * dvs[e]
                            out_v1[e, pl.ds(off, L)] = bv + fj[1][k] * dvs[e]

                for j in range(2):
                    pltpu.async_copy(
                        bufs[j], out_hbm.at[bb + j, pl.ds(e0, EC), :],
                        dma_sems[j])
                return _

            lax.fori_loop(0, BPW // 2, pair_body, None)
            return _

        lax.fori_loop(0, NEC, ec_body, None)

        for j in range(2):
            pltpu.make_async_copy(
                bufs[j],
                out_hbm.at[b0 + BPW - 2 + j,
                           pl.ds(pl.multiple_of((NEC - 1) * EC, EC), EC), :],
                dma_sems[j],
            ).wait()

    pl.run_scoped(run, pltpu.SemaphoreType.DMA, pltpu.SemaphoreType.DMA)


def kernel(fp, pair_emb, bit_emb, val_emb):
    H = D // 2
    base = (jnp.repeat(pair_emb, 2, axis=0)
            + jnp.tile(bit_emb, (H, 1))
            + val_emb[0][None, :])
    baset = base.T
    deltat = jnp.broadcast_to((val_emb[1] - val_emb[0])[:, None], (E, L))

    mesh = plsc.VectorSubcoreMesh(core_axis_name="c", subcore_axis_name="s")
    outt = pl.kernel(
        _sc_body,
        out_type=jax.ShapeDtypeStruct((B, E, D), jnp.float32),
        mesh=mesh,
        scratch_types=[
            pltpu.VMEM((EC, D), jnp.float32),
            pltpu.VMEM((EC, D), jnp.float32),
            pltpu.VMEM((EC, D), jnp.float32),
            pltpu.VMEM((2, D), jnp.int32),
            pltpu.VMEM((E, L), jnp.float32),
        ],
    )(fp, baset, deltat)
    return jnp.swapaxes(outt, 1, 2)


# speedup vs baseline: 1.1408x; 1.1408x over previous
import jax
import jax.numpy as jnp
from jax import lax
from jax.experimental import pallas as pl
from jax.experimental.pallas import tpu as pltpu
from jax.experimental.pallas import tpu_sc as plsc

B, D, E = 1024, 2048, 64
NC, NS, L = 2, 16, 16
NW = NC * NS
BPW = B // NW
DC = 512
NDC = D // DC
SUB = 128


def _sc_body(fp_hbm, baset_hbm, deltat_hbm, out_hbm,
             base_v, out_v, fp_v, fpf_v, deltat_v):
    wid = lax.axis_index("s") * NC + lax.axis_index("c")
    b0 = wid * BPW

    pltpu.sync_copy(deltat_hbm, deltat_v)

    for dc in range(NDC):
        pltpu.sync_copy(baset_hbm.at[:, pl.ds(dc * DC, DC)], base_v)

        def b_body(bi, _, dc=dc):
            b = b0 + bi
            pltpu.sync_copy(fp_hbm.at[b, pl.ds(dc * DC, DC)], fp_v)

            def conv_body(i, _):
                off = pl.multiple_of(i * L, L)
                fpf_v[pl.ds(off, L)] = fp_v[pl.ds(off, L)].astype(jnp.float32)
                return _

            lax.fori_loop(0, DC // L, conv_body, None)

            for sub in range(DC // SUB):
                fj = [fpf_v[pl.ds(sub * SUB + j * L, L)]
                      for j in range(SUB // L)]

                def e_body(e, _, sub=sub, fj=fj):
                    dv = deltat_v[e]
                    for j in range(SUB // L):
                        off = sub * SUB + j * L
                        out_v[e, pl.ds(off, L)] = (
                            base_v[e, pl.ds(off, L)] + fj[j] * dv)
                    return _

                lax.fori_loop(0, E, e_body, None)

            pltpu.sync_copy(out_v, out_hbm.at[b, :, pl.ds(dc * DC, DC)])
            return _

        lax.fori_loop(0, BPW, b_body, None)


def kernel(fp, pair_emb, bit_emb, val_emb):
    H = D // 2
    base = (jnp.repeat(pair_emb, 2, axis=0)
            + jnp.tile(bit_emb, (H, 1))
            + val_emb[0][None, :])
    baset = base.T
    deltat = jnp.broadcast_to((val_emb[1] - val_emb[0])[:, None], (E, L))

    mesh = plsc.VectorSubcoreMesh(core_axis_name="c", subcore_axis_name="s")
    outt = pl.kernel(
        _sc_body,
        out_type=jax.ShapeDtypeStruct((B, E, D), jnp.float32),
        mesh=mesh,
        scratch_types=[
            pltpu.VMEM((E, DC), jnp.float32),
            pltpu.VMEM((E, DC), jnp.float32),
            pltpu.VMEM((DC,), jnp.int32),
            pltpu.VMEM((DC,), jnp.float32),
            pltpu.VMEM((E, L), jnp.float32),
        ],
    )(fp, baset, deltat)
    return jnp.swapaxes(outt, 1, 2)
